# Initial kernel scaffold; baseline (speedup 1.0000x reference)
import jax
import jax.numpy as jnp
from jax.experimental import pallas as pl


def kernel(node_x, node_features, edge_index, batch, W_score, W_edge, W_anchor, W_out):
    raise NotImplementedError("write your pallas kernel here")


# speedup vs baseline: 11.1575x; 11.1575x over previous
import functools

import jax
import jax.numpy as jnp
from jax import lax
from jax.experimental import pallas as pl
from jax.experimental.pallas import tpu as pltpu
from jax.experimental.pallas import tpu_sc as plsc

N = 10000
NP = 10240
D = 128
K = 1000
KP = 1024
E = 320000
EPS = 1e-6
MININT = -(2 ** 31)

HI = jax.lax.Precision.HIGHEST

NTILES = 32
ET = E // NTILES
ETP = 10240
CFLAT = KP * KP
CSTRIPE = CFLAT // 16
CBUF = 8192


def _p1_body(nf_ref, w_ref, score_ref, nkl_ref):
    nf = nf_ref[...]
    w = w_ref[...]
    score = lax.dot_general(nf, w, (((1,), (0,)), ((), ())),
                            preferred_element_type=jnp.float32)
    score_ref[...] = score
    rid = lax.broadcasted_iota(jnp.int32, (NP, 1), 0)
    p = jax.nn.sigmoid(score)
    p = jnp.clip(p, EPS, 1.0 - EPS)
    t = p * jnp.log(2.0 * p) + (1.0 - p) * jnp.log(2.0 * (1.0 - p))
    nkl_ref[...] = jnp.reshape(jnp.sum(jnp.where(rid < N, t, 0.0)) / N, (1, 1))


def _lane_cumsum_incl(x):
    for k in (1, 2, 4, 8, 16, 32, 64):
        sh = pltpu.roll(x, k, axis=1)
        lid = lax.broadcasted_iota(jnp.int32, x.shape, 1)
        x = x + jnp.where(lid >= k, sh, 0)
    return x


def _prefix_excl(mask_i32):
    incl = _lane_cumsum_incl(mask_i32)
    rowsum = lax.slice(incl, (0, 127), (80, 128)).astype(jnp.float32)
    r0 = lax.broadcasted_iota(jnp.int32, (80, 80), 0)
    r1 = lax.broadcasted_iota(jnp.int32, (80, 80), 1)
    tri = (r0 > r1).astype(jnp.float32)
    offs = lax.dot_general(tri, rowsum, (((1,), (0,)), ((), ())),
                           preferred_element_type=jnp.float32)
    return incl - mask_i32 + offs.astype(jnp.int32)


def _p2a_body(score_ref, rank_ref):
    MINI = jnp.int32(MININT)
    score = score_ref[...]
    bits = lax.bitcast_convert_type(score, jnp.int32)
    sk = jnp.where(bits >= 0, bits, bits ^ jnp.int32(0x7FFFFFFF))
    rid = lax.broadcasted_iota(jnp.int32, (80, 128), 0)
    lid = lax.broadcasted_iota(jnp.int32, (80, 128), 1)
    flat = rid * 128 + lid
    sk = jnp.where(flat < N, sk, MINI)

    def body(i, tu):
        bit = lax.shift_left(jnp.int32(1), 31 - i)
        cand_u = tu | bit
        cand_s = cand_u ^ MINI
        cnt = jnp.sum((sk >= cand_s).astype(jnp.int32))
        return jnp.where(cnt >= K, cand_u, tu)

    tu = lax.fori_loop(0, 32, body, jnp.int32(0))
    ts = tu ^ MINI
    gt = sk > ts
    eq = sk == ts
    c_gt = jnp.sum(gt.astype(jnp.int32))
    need = K - c_gt
    pgt = _prefix_excl(gt.astype(jnp.int32))
    peq = _prefix_excl(eq.astype(jnp.int32))
    rank = jnp.where(gt, pgt, jnp.int32(-1))
    rank = jnp.where(eq & (peq < need), c_gt + peq, rank)
    rank_ref[...] = rank


def _p2b_body(rank_ref, pay_ref, anch_ref, akl_ref):
    i = pl.program_id(0)

    @pl.when(i == 0)
    def _():
        anch_ref[...] = jnp.zeros_like(anch_ref)

    r = rank_ref[...]
    lane = lax.broadcasted_iota(jnp.int32, (1024, KP), 1)
    m = (r == lane).astype(jnp.float32)
    anch_ref[...] += lax.dot_general(m, pay_ref[...], (((0,), (0,)), ((), ())),
                                     precision=HI,
                                     preferred_element_type=jnp.float32)

    @pl.when(i == pl.num_programs(0) - 1)
    def _():
        a = anch_ref[...]
        sc = a[:, 131:132]
        gate = jax.nn.sigmoid(sc)
        lane2 = lax.broadcasted_iota(jnp.int32, (KP, 256), 1)
        anch_ref[...] = jnp.where(lane2 < D, a * gate, a)
        rowk = lax.broadcasted_iota(jnp.int32, (KP, 1), 0)
        g = jnp.clip(gate, EPS, 1.0 - EPS)
        t = g * jnp.log(2.0 * g) + (1.0 - g) * jnp.log(2.0 * (1.0 - g))
        akl_ref[...] = jnp.reshape(jnp.sum(jnp.where(rowk < K, t, 0.0)) / K, (1, 1))


def _p3_body(nx_ref, ax_ref, asg_ref):
    x = nx_ref[...]
    a = ax_ref[...]
    xy = lax.dot_general(x, a, (((1,), (0,)), ((), ())),
                         preferred_element_type=jnp.float32)
    sx = jnp.sum(x * x, axis=1, keepdims=True)
    sa = jnp.sum(a * a, axis=0, keepdims=True)
    d2 = (sx + sa) - 2.0 * xy
    lane = lax.broadcasted_iota(jnp.int32, d2.shape, 1)
    d2 = d2 + jnp.where(lane >= K, jnp.float32(3.0e38), jnp.float32(0.0))
    mn = jnp.min(d2, axis=1, keepdims=True)
    idx = jnp.min(jnp.where(d2 == mn, lane, jnp.int32(2 ** 30)),
                  axis=1, keepdims=True)
    asg_ref[...] = idx


def _p4_body(asg_ref, nf_ref, nx_ref, ax4_ref, we_ref, af0_ref, agg_ref, af1_ref):
    i = pl.program_id(0)

    @pl.when(i == 0)
    def _():
        agg_ref[...] = jnp.zeros_like(agg_ref)

    asg = asg_ref[...]
    lane = lax.broadcasted_iota(jnp.int32, (1024, KP), 1)
    m = (asg == lane).astype(jnp.float32)
    ue = lax.dot_general(ax4_ref[...], we_ref[...], (((1,), (0,)), ((), ())),
                         preferred_element_type=jnp.float32)
    ua = lax.dot_general(m, ue, (((1,), (0,)), ((), ())), precision=HI,
                         preferred_element_type=jnp.float32)
    ve = lax.dot_general(nx_ref[...], we_ref[...], (((1,), (0,)), ((), ())),
                         preferred_element_type=jnp.float32)
    msg = nf_ref[...] + jax.nn.relu(ua - ve)
    rid = lax.broadcasted_iota(jnp.int32, (1024, 1), 0)
    valid = (i * 1024 + rid) < N
    lane128 = lax.broadcasted_iota(jnp.int32, (1024, D), 1)
    ones = jnp.where(lane128 == 0, 1.0, 0.0)
    msgext = jnp.concatenate(
        [jnp.where(valid, msg, 0.0), jnp.where(valid, ones, 0.0)], axis=1)
    agg_ref[...] += lax.dot_general(m, msgext, (((0,), (0,)), ((), ())),
                                    precision=HI,
                                    preferred_element_type=jnp.float32)

    @pl.when(i == pl.num_programs(0) - 1)
    def _():
        agg = agg_ref[...]
        cnt = agg[:, D:D + 1]
        af1_ref[...] = af0_ref[...] + agg[:, :D] / jnp.maximum(cnt, 1.0)


def _p5_body(c0_ref, c1_ref, af1_ref, ax4_ref, we_ref, wa_ref, af2_ref, t2_ref, ue_ref):
    af1 = af1_ref[...]
    term1 = lax.dot_general(c0_ref[...] + c1_ref[...], af1,
                            (((1,), (0,)), ((), ())), precision=HI,
                            preferred_element_type=jnp.float32)
    ue_ref[...] = lax.dot_general(ax4_ref[...], we_ref[...],
                                  (((1,), (0,)), ((), ())),
                                  preferred_element_type=jnp.float32)

    def dblock(j, _):
        ued = ue_ref[pl.ds(j * 8, 8), :]

        def schunk(mm, acc):
            ue_sc = ue_ref[pl.ds(mm * 128, 128), :]
            w = (c0_ref[pl.ds(j * 8, 8), pl.ds(mm * 128, 128)]
                 + c1_ref[pl.ds(j * 8, 8), pl.ds(mm * 128, 128)])
            sidx = lax.broadcasted_iota(jnp.int32, (8, 128), 1) + mm * 128
            w = jnp.where(sidx < K, w, 0.0)
            t = jax.nn.relu(ue_sc[None, :, :] - ued[:, None, :])
            return acc + jnp.sum(t * w[:, :, None], axis=1)

        acc = lax.fori_loop(0, KP // 128, schunk, jnp.zeros((8, D), jnp.float32))
        t2_ref[pl.ds(j * 8, 8), :] = acc
        return 0

    lax.fori_loop(0, KP // 8, dblock, 0)
    x = af1 + term1 + t2_ref[...]
    af2_ref[...] = jax.nn.relu(
        lax.dot_general(x, wa_ref[...], (((1,), (0,)), ((), ())),
                        preferred_element_type=jnp.float32))


def _p6_body(asg_ref, nf_ref, nx_ref, af2_ref, ax4_ref, we_ref, wo_ref, out_ref):
    asg = asg_ref[...]
    lane = lax.broadcasted_iota(jnp.int32, (1024, KP), 1)
    m = (asg == lane).astype(jnp.float32)
    ue = lax.dot_general(ax4_ref[...], we_ref[...], (((1,), (0,)), ((), ())),
                         preferred_element_type=jnp.float32)
    pack = jnp.concatenate([af2_ref[...], ue], axis=1)
    g = lax.dot_general(m, pack, (((1,), (0,)), ((), ())), precision=HI,
                        preferred_element_type=jnp.float32)
    ve = lax.dot_general(nx_ref[...], we_ref[...], (((1,), (0,)), ((), ())),
                         preferred_element_type=jnp.float32)
    gathered = g[:, :D] + jax.nn.relu(ve - g[:, D:])
    upd = jax.nn.relu(
        lax.dot_general(gathered, wo_ref[...], (((1,), (0,)), ((), ())),
                        preferred_element_type=jnp.float32))
    out_ref[...] = nf_ref[...] + upd


def _sc_hist_body(es_hbm, ed_hbm, asg_hbm, out_hbm,
                  src_v, dst_v, asg_v, pair_v, ones_v, buf_v, c_sh):
    c = lax.axis_index("c")
    s = lax.axis_index("s")
    wid = c * 16 + s

    def zb(i, _):
        buf_v[pl.ds(i * 16, 16)] = jnp.zeros((16,), jnp.float32)
        return 0
    lax.fori_loop(0, CBUF // 16, zb, 0)

    def zcp(t, _):
        pltpu.sync_copy(buf_v, c_sh.at[pl.ds(s * CSTRIPE + t * CBUF, CBUF)])
        return 0
    lax.fori_loop(0, CSTRIPE // CBUF, zcp, 0)
    plsc.subcore_barrier()

    base = wid * ET
    pltpu.sync_copy(es_hbm.at[pl.ds(base, ET)], src_v)
    pltpu.sync_copy(ed_hbm.at[pl.ds(base, ET)], dst_v)
    pltpu.sync_copy(asg_hbm, asg_v)

    def step(i, _):
        valid = i < (ET // 16)
        off = jnp.where(valid, i * 16, 0)
        sv = src_v[pl.ds(off, 16)]
        dv = dst_v[pl.ds(off, 16)]
        a_s = plsc.load_gather(asg_v, [sv])
        a_d = plsc.load_gather(asg_v, [dv])
        pair = jnp.where(valid, a_d * KP + a_s, jnp.int32(CFLAT - 1))
        ones = jnp.where(valid, jnp.float32(1.0), jnp.float32(0.0))
        ones16 = jnp.full((16,), 0.0, jnp.float32) + ones
        pair_v[pl.ds(i * 16, 16)] = pair
        ones_v[pl.ds(i * 16, 16)] = ones16
        return 0

    lax.fori_loop(0, ETP // 16, step, 0)
    pltpu.sync_copy(ones_v, c_sh.at[pair_v], add=True)
    plsc.subcore_barrier()

    def xcp(t, _):
        pltpu.sync_copy(c_sh.at[pl.ds(s * CSTRIPE + t * CBUF, CBUF)], buf_v)
        pltpu.sync_copy(buf_v, out_hbm.at[c, pl.ds(s * CSTRIPE + t * CBUF, CBUF)])
        return 0
    lax.fori_loop(0, CSTRIPE // CBUF, xcp, 0)


def _sc_hist(edge_src, edge_dst, assign_flat):
    mesh = plsc.VectorSubcoreMesh(core_axis_name="c", subcore_axis_name="s")
    f = pl.kernel(
        _sc_hist_body,
        out_type=jax.ShapeDtypeStruct((2, CFLAT), jnp.float32),
        mesh=mesh,
        compiler_params=pltpu.CompilerParams(needs_layout_passes=False),
        scratch_types=[
            pltpu.VMEM((ET,), jnp.int32),
            pltpu.VMEM((ET,), jnp.int32),
            pltpu.VMEM((NP,), jnp.int32),
            pltpu.VMEM((ETP,), jnp.int32),
            pltpu.VMEM((ETP,), jnp.float32),
            pltpu.VMEM((CBUF,), jnp.float32),
            pltpu.VMEM_SHARED((CFLAT,), jnp.float32),
        ],
    )
    return f(edge_src, edge_dst, assign_flat)


def kernel(node_x, node_features, edge_index, batch, W_score, W_edge, W_anchor, W_out):
    f32 = jnp.float32
    nf_pad = jnp.pad(node_features, ((0, NP - N), (0, 0)))
    nx4 = jnp.pad(node_x, ((0, NP - N), (0, 1)))
    we4 = jnp.pad(W_edge, ((0, 1), (0, 0)))
    es = edge_index[0].astype(jnp.int32)
    ed = edge_index[1].astype(jnp.int32)

    score, nkl = pl.pallas_call(
        _p1_body,
        out_shape=[jax.ShapeDtypeStruct((NP, 1), f32),
                   jax.ShapeDtypeStruct((1, 1), f32)],
    )(nf_pad, W_score)

    rank80 = pl.pallas_call(
        _p2a_body,
        out_shape=jax.ShapeDtypeStruct((80, 128), jnp.int32),
    )(score.reshape(80, 128))
    rank2 = rank80.reshape(NP, 1)

    payload = jnp.concatenate(
        [nf_pad, nx4[:, :3], score, jnp.zeros((NP, 124), f32)], axis=1)
    anchors, akl = pl.pallas_call(
        _p2b_body,
        grid=(NP // 1024,),
        in_specs=[pl.BlockSpec((1024, 1), lambda i: (i, 0)),
                  pl.BlockSpec((1024, 256), lambda i: (i, 0))],
        out_specs=[pl.BlockSpec((KP, 256), lambda i: (0, 0)),
                   pl.BlockSpec((1, 1), lambda i: (0, 0))],
        out_shape=[jax.ShapeDtypeStruct((KP, 256), f32),
                   jax.ShapeDtypeStruct((1, 1), f32)],
    )(rank2, payload)

    af0 = anchors[:, :D]
    ax4 = jnp.concatenate([anchors[:, D:D + 3], jnp.zeros((KP, 1), f32)], axis=1)
    axT4 = ax4.T

    assign = pl.pallas_call(
        _p3_body,
        grid=(NP // 2048,),
        in_specs=[pl.BlockSpec((2048, 4), lambda i: (i, 0)),
                  pl.BlockSpec((4, KP), lambda i: (0, 0))],
        out_specs=pl.BlockSpec((2048, 1), lambda i: (i, 0)),
        out_shape=jax.ShapeDtypeStruct((NP, 1), jnp.int32),
    )(nx4, axT4)

    c2 = _sc_hist(es, ed, assign.reshape(NP))
    c0 = c2[0].reshape(KP, KP)
    c1 = c2[1].reshape(KP, KP)

    _, af1 = pl.pallas_call(
        _p4_body,
        grid=(NP // 1024,),
        in_specs=[pl.BlockSpec((1024, 1), lambda i: (i, 0)),
                  pl.BlockSpec((1024, D), lambda i: (i, 0)),
                  pl.BlockSpec((1024, 4), lambda i: (i, 0)),
                  pl.BlockSpec((KP, 4), lambda i: (0, 0)),
                  pl.BlockSpec((4, D), lambda i: (0, 0)),
                  pl.BlockSpec((KP, D), lambda i: (0, 0))],
        out_specs=[pl.BlockSpec((KP, 2 * D), lambda i: (0, 0)),
                   pl.BlockSpec((KP, D), lambda i: (0, 0))],
        out_shape=[jax.ShapeDtypeStruct((KP, 2 * D), f32),
                   jax.ShapeDtypeStruct((KP, D), f32)],
    )(assign, nf_pad, nx4, ax4, we4, af0)

    af2 = pl.pallas_call(
        _p5_body,
        out_shape=jax.ShapeDtypeStruct((KP, D), f32),
        scratch_shapes=[pltpu.VMEM((KP, D), f32), pltpu.VMEM((KP, D), f32)],
    )(c0, c1, af1, ax4, we4, W_anchor)

    nf_out = pl.pallas_call(
        _p6_body,
        grid=(NP // 1024,),
        in_specs=[pl.BlockSpec((1024, 1), lambda i: (i, 0)),
                  pl.BlockSpec((1024, D), lambda i: (i, 0)),
                  pl.BlockSpec((1024, 4), lambda i: (i, 0)),
                  pl.BlockSpec((KP, D), lambda i: (0, 0)),
                  pl.BlockSpec((KP, 4), lambda i: (0, 0)),
                  pl.BlockSpec((4, D), lambda i: (0, 0)),
                  pl.BlockSpec((D, D), lambda i: (0, 0))],
        out_specs=pl.BlockSpec((1024, D), lambda i: (i, 0)),
        out_shape=jax.ShapeDtypeStruct((NP, D), f32),
    )(assign, nf_pad, nx4, af2, ax4, we4, W_out)

    return (nf_out[:N], akl[0, 0], nkl[0, 0])


# speedup vs baseline: 13.3920x; 1.2003x over previous
import functools

import jax
import jax.numpy as jnp
from jax import lax
from jax.experimental import pallas as pl
from jax.experimental.pallas import tpu as pltpu
from jax.experimental.pallas import tpu_sc as plsc

N = 10000
NP = 10240
D = 128
K = 1000
KP = 1024
E = 320000
EPS = 1e-6
MININT = -(2 ** 31)

HI = jax.lax.Precision.HIGHEST


def _dot2(m, x, dims):
    hi = x.astype(jnp.bfloat16).astype(jnp.float32)
    lo = x - hi
    return (lax.dot_general(m, hi, dims, preferred_element_type=jnp.float32)
            + lax.dot_general(m, lo, dims, preferred_element_type=jnp.float32))

NTILES = 32
ET = E // NTILES
ETP = 10240
CFLAT = KP * KP
CSTRIPE = CFLAT // 16
CBUF = 8192


def _p1_body(nf_ref, w_ref, score_ref, nkl_ref):
    nf = nf_ref[...]
    w = w_ref[...]
    score = lax.dot_general(nf, w, (((1,), (0,)), ((), ())),
                            preferred_element_type=jnp.float32)
    score_ref[...] = score
    rid = lax.broadcasted_iota(jnp.int32, (NP, 1), 0)
    p = jax.nn.sigmoid(score)
    p = jnp.clip(p, EPS, 1.0 - EPS)
    t = p * jnp.log(2.0 * p) + (1.0 - p) * jnp.log(2.0 * (1.0 - p))
    nkl_ref[...] = jnp.reshape(jnp.sum(jnp.where(rid < N, t, 0.0)) / N, (1, 1))


def _lane_cumsum_incl(x):
    for k in (1, 2, 4, 8, 16, 32, 64):
        sh = pltpu.roll(x, k, axis=1)
        lid = lax.broadcasted_iota(jnp.int32, x.shape, 1)
        x = x + jnp.where(lid >= k, sh, 0)
    return x


def _prefix_excl(mask_i32):
    incl = _lane_cumsum_incl(mask_i32)
    rowsum = lax.slice(incl, (0, 127), (80, 128)).astype(jnp.float32)
    r0 = lax.broadcasted_iota(jnp.int32, (80, 80), 0)
    r1 = lax.broadcasted_iota(jnp.int32, (80, 80), 1)
    tri = (r0 > r1).astype(jnp.float32)
    offs = lax.dot_general(tri, rowsum, (((1,), (0,)), ((), ())),
                           preferred_element_type=jnp.float32)
    return incl - mask_i32 + offs.astype(jnp.int32)


def _p2a_body(score_ref, rank_ref):
    MINI = jnp.int32(MININT)
    score = score_ref[...]
    bits = lax.bitcast_convert_type(score, jnp.int32)
    sk = jnp.where(bits >= 0, bits, bits ^ jnp.int32(0x7FFFFFFF))
    rid = lax.broadcasted_iota(jnp.int32, (80, 128), 0)
    lid = lax.broadcasted_iota(jnp.int32, (80, 128), 1)
    flat = rid * 128 + lid
    sk = jnp.where(flat < N, sk, MINI)

    def body(i, tu):
        bit = lax.shift_left(jnp.int32(1), 31 - i)
        cand_u = tu | bit
        cand_s = cand_u ^ MINI
        cnt = jnp.sum((sk >= cand_s).astype(jnp.int32))
        return jnp.where(cnt >= K, cand_u, tu)

    tu = lax.fori_loop(0, 32, body, jnp.int32(0))
    ts = tu ^ MINI
    gt = sk > ts
    eq = sk == ts
    c_gt = jnp.sum(gt.astype(jnp.int32))
    need = K - c_gt
    pgt = _prefix_excl(gt.astype(jnp.int32))
    peq = _prefix_excl(eq.astype(jnp.int32))
    rank = jnp.where(gt, pgt, jnp.int32(-1))
    rank = jnp.where(eq & (peq < need), c_gt + peq, rank)
    rank_ref[...] = rank


def _p2b_body(rank_ref, nf_ref, payx_ref, anch_ref, anchx_ref, akl_ref):
    i = pl.program_id(0)

    @pl.when(i == 0)
    def _():
        anch_ref[...] = jnp.zeros_like(anch_ref)
        anchx_ref[...] = jnp.zeros_like(anchx_ref)

    r = rank_ref[...]
    lane = lax.broadcasted_iota(jnp.int32, (1024, KP), 1)
    m = (r == lane).astype(jnp.float32)
    dims = (((0,), (0,)), ((), ()))
    anch_ref[...] += _dot2(m, nf_ref[...], dims)
    anchx_ref[...] += lax.dot_general(m, payx_ref[...], dims, precision=HI,
                                      preferred_element_type=jnp.float32)

    @pl.when(i == pl.num_programs(0) - 1)
    def _():
        sc = anchx_ref[:, 3:4]
        gate = jax.nn.sigmoid(sc)
        anch_ref[...] = anch_ref[...] * gate
        rowk = lax.broadcasted_iota(jnp.int32, (KP, 1), 0)
        g = jnp.clip(gate, EPS, 1.0 - EPS)
        t = g * jnp.log(2.0 * g) + (1.0 - g) * jnp.log(2.0 * (1.0 - g))
        akl_ref[...] = jnp.reshape(jnp.sum(jnp.where(rowk < K, t, 0.0)) / K, (1, 1))


def _p3_body(nx_ref, ax_ref, asg_ref):
    x = nx_ref[...]
    a = ax_ref[...]
    xy = lax.dot_general(x, a, (((1,), (0,)), ((), ())),
                         preferred_element_type=jnp.float32)
    sx = jnp.sum(x * x, axis=1, keepdims=True)
    sa = jnp.sum(a * a, axis=0, keepdims=True)
    d2 = (sx + sa) - 2.0 * xy
    lane = lax.broadcasted_iota(jnp.int32, d2.shape, 1)
    d2 = d2 + jnp.where(lane >= K, jnp.float32(3.0e38), jnp.float32(0.0))
    mn = jnp.min(d2, axis=1, keepdims=True)
    idx = jnp.min(jnp.where(d2 == mn, lane, jnp.int32(2 ** 30)),
                  axis=1, keepdims=True)
    asg_ref[...] = idx


def _p4_body(asg_ref, nf_ref, nx_ref, ax4_ref, we_ref, af0_ref,
             agg_ref, cnt_ref, af1_ref):
    i = pl.program_id(0)

    @pl.when(i == 0)
    def _():
        agg_ref[...] = jnp.zeros_like(agg_ref)
        cnt_ref[...] = jnp.zeros_like(cnt_ref)

    asg = asg_ref[...]
    lane = lax.broadcasted_iota(jnp.int32, (1024, KP), 1)
    m = (asg == lane).astype(jnp.float32)
    ue = lax.dot_general(ax4_ref[...], we_ref[...], (((1,), (0,)), ((), ())),
                         preferred_element_type=jnp.float32)
    ua = _dot2(m, ue, (((1,), (0,)), ((), ())))
    ve = lax.dot_general(nx_ref[...], we_ref[...], (((1,), (0,)), ((), ())),
                         preferred_element_type=jnp.float32)
    msg = nf_ref[...] + jax.nn.relu(ua - ve)
    rid = lax.broadcasted_iota(jnp.int32, (1024, 1), 0)
    valid = (i * 1024 + rid) < N
    dims = (((0,), (0,)), ((), ()))
    agg_ref[...] += _dot2(m, jnp.where(valid, msg, 0.0), dims)
    lane8 = lax.broadcasted_iota(jnp.int32, (1024, 8), 1)
    vcol = jnp.where((lane8 == 0) & valid, 1.0, 0.0)
    cnt_ref[...] += lax.dot_general(m, vcol, dims,
                                    preferred_element_type=jnp.float32)

    @pl.when(i == pl.num_programs(0) - 1)
    def _():
        cnt = cnt_ref[:, 0:1]
        af1_ref[...] = af0_ref[...] + agg_ref[...] / jnp.maximum(cnt, 1.0)


def _p5_body(c0_ref, c1_ref, af1_ref, ax4_ref, we_ref, wa_ref, af2_ref, t2_ref, ue_ref):
    af1 = af1_ref[...]
    term1 = lax.dot_general(c0_ref[...] + c1_ref[...], af1,
                            (((1,), (0,)), ((), ())), precision=HI,
                            preferred_element_type=jnp.float32)
    ue_ref[...] = lax.dot_general(ax4_ref[...], we_ref[...],
                                  (((1,), (0,)), ((), ())),
                                  preferred_element_type=jnp.float32)

    def dblock(j, _):
        ued = ue_ref[pl.ds(j * 8, 8), :]

        def schunk(mm, acc):
            ue_sc = ue_ref[pl.ds(mm * 128, 128), :]
            w = (c0_ref[pl.ds(j * 8, 8), pl.ds(mm * 128, 128)]
                 + c1_ref[pl.ds(j * 8, 8), pl.ds(mm * 128, 128)])
            sidx = lax.broadcasted_iota(jnp.int32, (8, 128), 1) + mm * 128
            w = jnp.where(sidx < K, w, 0.0)
            t = jax.nn.relu(ue_sc[None, :, :] - ued[:, None, :])
            return acc + jnp.sum(t * w[:, :, None], axis=1)

        acc = lax.fori_loop(0, KP // 128, schunk, jnp.zeros((8, D), jnp.float32))
        t2_ref[pl.ds(j * 8, 8), :] = acc
        return 0

    lax.fori_loop(0, KP // 8, dblock, 0)
    x = af1 + term1 + t2_ref[...]
    af2_ref[...] = jax.nn.relu(
        lax.dot_general(x, wa_ref[...], (((1,), (0,)), ((), ())),
                        preferred_element_type=jnp.float32))


def _p6_body(asg_ref, nf_ref, nx_ref, af2_ref, ax4_ref, we_ref, wo_ref, out_ref):
    asg = asg_ref[...]
    lane = lax.broadcasted_iota(jnp.int32, (1024, KP), 1)
    m = (asg == lane).astype(jnp.float32)
    ue = lax.dot_general(ax4_ref[...], we_ref[...], (((1,), (0,)), ((), ())),
                         preferred_element_type=jnp.float32)
    pack = jnp.concatenate([af2_ref[...], ue], axis=1)
    g = _dot2(m, pack, (((1,), (0,)), ((), ())))
    ve = lax.dot_general(nx_ref[...], we_ref[...], (((1,), (0,)), ((), ())),
                         preferred_element_type=jnp.float32)
    gathered = g[:, :D] + jax.nn.relu(ve - g[:, D:])
    upd = jax.nn.relu(
        lax.dot_general(gathered, wo_ref[...], (((1,), (0,)), ((), ())),
                        preferred_element_type=jnp.float32))
    out_ref[...] = nf_ref[...] + upd


def _sc_hist_body(es_hbm, ed_hbm, asg_hbm, out_hbm,
                  src_v, dst_v, asg_v, pair_v, ones_v, buf_v, c_sh):
    c = lax.axis_index("c")
    s = lax.axis_index("s")
    wid = c * 16 + s

    def zb(i, _):
        buf_v[pl.ds(i * 16, 16)] = jnp.zeros((16,), jnp.float32)
        return 0
    lax.fori_loop(0, CBUF // 16, zb, 0)

    def zcp(t, _):
        pltpu.sync_copy(buf_v, c_sh.at[pl.ds(s * CSTRIPE + t * CBUF, CBUF)])
        return 0
    lax.fori_loop(0, CSTRIPE // CBUF, zcp, 0)
    plsc.subcore_barrier()

    base = wid * ET
    pltpu.sync_copy(es_hbm.at[pl.ds(base, ET)], src_v)
    pltpu.sync_copy(ed_hbm.at[pl.ds(base, ET)], dst_v)
    pltpu.sync_copy(asg_hbm, asg_v)

    def step(i, _):
        valid = i < (ET // 16)
        off = jnp.where(valid, i * 16, 0)
        sv = src_v[pl.ds(off, 16)]
        dv = dst_v[pl.ds(off, 16)]
        a_s = plsc.load_gather(asg_v, [sv])
        a_d = plsc.load_gather(asg_v, [dv])
        pair = jnp.where(valid, a_d * KP + a_s, jnp.int32(CFLAT - 1))
        ones = jnp.where(valid, jnp.float32(1.0), jnp.float32(0.0))
        ones16 = jnp.full((16,), 0.0, jnp.float32) + ones
        pair_v[pl.ds(i * 16, 16)] = pair
        ones_v[pl.ds(i * 16, 16)] = ones16
        return 0

    lax.fori_loop(0, ETP // 16, step, 0)
    pltpu.sync_copy(ones_v, c_sh.at[pair_v], add=True)
    plsc.subcore_barrier()

    def xcp(t, _):
        pltpu.sync_copy(c_sh.at[pl.ds(s * CSTRIPE + t * CBUF, CBUF)], buf_v)
        pltpu.sync_copy(buf_v, out_hbm.at[c, pl.ds(s * CSTRIPE + t * CBUF, CBUF)])
        return 0
    lax.fori_loop(0, CSTRIPE // CBUF, xcp, 0)


def _sc_hist(edge_src, edge_dst, assign_flat):
    mesh = plsc.VectorSubcoreMesh(core_axis_name="c", subcore_axis_name="s")
    f = pl.kernel(
        _sc_hist_body,
        out_type=jax.ShapeDtypeStruct((2, CFLAT), jnp.float32),
        mesh=mesh,
        compiler_params=pltpu.CompilerParams(needs_layout_passes=False),
        scratch_types=[
            pltpu.VMEM((ET,), jnp.int32),
            pltpu.VMEM((ET,), jnp.int32),
            pltpu.VMEM((NP,), jnp.int32),
            pltpu.VMEM((ETP,), jnp.int32),
            pltpu.VMEM((ETP,), jnp.float32),
            pltpu.VMEM((CBUF,), jnp.float32),
            pltpu.VMEM_SHARED((CFLAT,), jnp.float32),
        ],
    )
    return f(edge_src, edge_dst, assign_flat)


def kernel(node_x, node_features, edge_index, batch, W_score, W_edge, W_anchor, W_out):
    f32 = jnp.float32
    nf_pad = jnp.pad(node_features, ((0, NP - N), (0, 0)))
    nx4 = jnp.pad(node_x, ((0, NP - N), (0, 1)))
    we4 = jnp.pad(W_edge, ((0, 1), (0, 0)))
    es = edge_index[0].astype(jnp.int32)
    ed = edge_index[1].astype(jnp.int32)

    score, nkl = pl.pallas_call(
        _p1_body,
        out_shape=[jax.ShapeDtypeStruct((NP, 1), f32),
                   jax.ShapeDtypeStruct((1, 1), f32)],
    )(nf_pad, W_score)

    rank80 = pl.pallas_call(
        _p2a_body,
        out_shape=jax.ShapeDtypeStruct((80, 128), jnp.int32),
    )(score.reshape(80, 128))
    rank2 = rank80.reshape(NP, 1)

    payx = jnp.concatenate([nx4[:, :3], score, jnp.zeros((NP, 4), f32)], axis=1)
    af0, anchx, akl = pl.pallas_call(
        _p2b_body,
        grid=(NP // 1024,),
        in_specs=[pl.BlockSpec((1024, 1), lambda i: (i, 0)),
                  pl.BlockSpec((1024, D), lambda i: (i, 0)),
                  pl.BlockSpec((1024, 8), lambda i: (i, 0))],
        out_specs=[pl.BlockSpec((KP, D), lambda i: (0, 0)),
                   pl.BlockSpec((KP, 8), lambda i: (0, 0)),
                   pl.BlockSpec((1, 1), lambda i: (0, 0))],
        out_shape=[jax.ShapeDtypeStruct((KP, D), f32),
                   jax.ShapeDtypeStruct((KP, 8), f32),
                   jax.ShapeDtypeStruct((1, 1), f32)],
    )(rank2, nf_pad, payx)

    ax4 = jnp.concatenate([anchx[:, :3], jnp.zeros((KP, 1), f32)], axis=1)
    axT4 = ax4.T

    assign = pl.pallas_call(
        _p3_body,
        grid=(NP // 2048,),
        in_specs=[pl.BlockSpec((2048, 4), lambda i: (i, 0)),
                  pl.BlockSpec((4, KP), lambda i: (0, 0))],
        out_specs=pl.BlockSpec((2048, 1), lambda i: (i, 0)),
        out_shape=jax.ShapeDtypeStruct((NP, 1), jnp.int32),
    )(nx4, axT4)

    c2 = _sc_hist(es, ed, assign.reshape(NP))
    c0 = c2[0].reshape(KP, KP)
    c1 = c2[1].reshape(KP, KP)

    _, _, af1 = pl.pallas_call(
        _p4_body,
        grid=(NP // 1024,),
        in_specs=[pl.BlockSpec((1024, 1), lambda i: (i, 0)),
                  pl.BlockSpec((1024, D), lambda i: (i, 0)),
                  pl.BlockSpec((1024, 4), lambda i: (i, 0)),
                  pl.BlockSpec((KP, 4), lambda i: (0, 0)),
                  pl.BlockSpec((4, D), lambda i: (0, 0)),
                  pl.BlockSpec((KP, D), lambda i: (0, 0))],
        out_specs=[pl.BlockSpec((KP, D), lambda i: (0, 0)),
                   pl.BlockSpec((KP, 8), lambda i: (0, 0)),
                   pl.BlockSpec((KP, D), lambda i: (0, 0))],
        out_shape=[jax.ShapeDtypeStruct((KP, D), f32),
                   jax.ShapeDtypeStruct((KP, 8), f32),
                   jax.ShapeDtypeStruct((KP, D), f32)],
    )(assign, nf_pad, nx4, ax4, we4, af0)

    af2 = pl.pallas_call(
        _p5_body,
        out_shape=jax.ShapeDtypeStruct((KP, D), f32),
        scratch_shapes=[pltpu.VMEM((KP, D), f32), pltpu.VMEM((KP, D), f32)],
    )(c0, c1, af1, ax4, we4, W_anchor)

    nf_out = pl.pallas_call(
        _p6_body,
        grid=(NP // 1024,),
        in_specs=[pl.BlockSpec((1024, 1), lambda i: (i, 0)),
                  pl.BlockSpec((1024, D), lambda i: (i, 0)),
                  pl.BlockSpec((1024, 4), lambda i: (i, 0)),
                  pl.BlockSpec((KP, D), lambda i: (0, 0)),
                  pl.BlockSpec((KP, 4), lambda i: (0, 0)),
                  pl.BlockSpec((4, D), lambda i: (0, 0)),
                  pl.BlockSpec((D, D), lambda i: (0, 0))],
        out_specs=pl.BlockSpec((1024, D), lambda i: (i, 0)),
        out_shape=jax.ShapeDtypeStruct((NP, D), f32),
    )(assign, nf_pad, nx4, af2, ax4, we4, W_out)

    return (nf_out[:N], akl[0, 0], nkl[0, 0])


# speedup vs baseline: 13.6241x; 1.0173x over previous
import functools

import jax
import jax.numpy as jnp
from jax import lax
from jax.experimental import pallas as pl
from jax.experimental.pallas import tpu as pltpu
from jax.experimental.pallas import tpu_sc as plsc

N = 10000
NP = 10240
D = 128
K = 1000
KP = 1024
E = 320000
EPS = 1e-6
MININT = -(2 ** 31)

HI = jax.lax.Precision.HIGHEST


def _dot2(m, x, dims):
    hi = x.astype(jnp.bfloat16).astype(jnp.float32)
    lo = x - hi
    return (lax.dot_general(m, hi, dims, preferred_element_type=jnp.float32)
            + lax.dot_general(m, lo, dims, preferred_element_type=jnp.float32))

NTILES = 32
ET = E // NTILES
ETP = 10240
CFLAT = KP * KP
CSTRIPE = CFLAT // 16
CBUF = 8192


def _p1_body(nf_ref, w_ref, score_ref, rank_ref, nkl_ref):
    nf = nf_ref[...]
    w = w_ref[...]
    score = lax.dot_general(nf, w, (((1,), (0,)), ((), ())),
                            preferred_element_type=jnp.float32)
    score_ref[...] = score
    rid = lax.broadcasted_iota(jnp.int32, (NP, 1), 0)
    p = jax.nn.sigmoid(score)
    p = jnp.clip(p, EPS, 1.0 - EPS)
    t = p * jnp.log(2.0 * p) + (1.0 - p) * jnp.log(2.0 * (1.0 - p))
    nkl_ref[...] = jnp.reshape(jnp.sum(jnp.where(rid < N, t, 0.0)) / N, (1, 1))
    _p2a_select(jnp.reshape(score, (80, 128)), rank_ref)


def _lane_cumsum_incl(x):
    for k in (1, 2, 4, 8, 16, 32, 64):
        sh = pltpu.roll(x, k, axis=1)
        lid = lax.broadcasted_iota(jnp.int32, x.shape, 1)
        x = x + jnp.where(lid >= k, sh, 0)
    return x


def _prefix_excl(mask_i32):
    incl = _lane_cumsum_incl(mask_i32)
    rowsum = lax.slice(incl, (0, 127), (80, 128)).astype(jnp.float32)
    r0 = lax.broadcasted_iota(jnp.int32, (80, 80), 0)
    r1 = lax.broadcasted_iota(jnp.int32, (80, 80), 1)
    tri = (r0 > r1).astype(jnp.float32)
    offs = lax.dot_general(tri, rowsum, (((1,), (0,)), ((), ())),
                           preferred_element_type=jnp.float32)
    return incl - mask_i32 + offs.astype(jnp.int32)


def _p2a_select(score, rank_ref):
    MINI = jnp.int32(MININT)
    bits = lax.bitcast_convert_type(score, jnp.int32)
    sk = jnp.where(bits >= 0, bits, bits ^ jnp.int32(0x7FFFFFFF))
    rid = lax.broadcasted_iota(jnp.int32, (80, 128), 0)
    lid = lax.broadcasted_iota(jnp.int32, (80, 128), 1)
    flat = rid * 128 + lid
    sk = jnp.where(flat < N, sk, MINI)

    def body(i, tu):
        bit = lax.shift_left(jnp.int32(1), 31 - i)
        cand_u = tu | bit
        cand_s = cand_u ^ MINI
        cnt = jnp.sum((sk >= cand_s).astype(jnp.int32))
        return jnp.where(cnt >= K, cand_u, tu)

    tu = lax.fori_loop(0, 32, body, jnp.int32(0))
    ts = tu ^ MINI
    gt = sk > ts
    eq = sk == ts
    c_gt = jnp.sum(gt.astype(jnp.int32))
    need = K - c_gt
    pgt = _prefix_excl(gt.astype(jnp.int32))
    peq = _prefix_excl(eq.astype(jnp.int32))
    rank = jnp.where(gt, pgt, jnp.int32(-1))
    rank = jnp.where(eq & (peq < need), c_gt + peq, rank)
    rank_ref[...] = rank


def _p2b_body(rank_ref, nf_ref, payx_ref, anch_ref, anchx_ref, akl_ref):
    i = pl.program_id(0)

    @pl.when(i == 0)
    def _():
        anch_ref[...] = jnp.zeros_like(anch_ref)
        anchx_ref[...] = jnp.zeros_like(anchx_ref)

    r = rank_ref[...]
    lane = lax.broadcasted_iota(jnp.int32, (1024, KP), 1)
    m = (r == lane).astype(jnp.float32)
    dims = (((0,), (0,)), ((), ()))
    anch_ref[...] += _dot2(m, nf_ref[...], dims)
    anchx_ref[...] += lax.dot_general(m, payx_ref[...], dims, precision=HI,
                                      preferred_element_type=jnp.float32)

    @pl.when(i == pl.num_programs(0) - 1)
    def _():
        sc = anchx_ref[:, 3:4]
        gate = jax.nn.sigmoid(sc)
        anch_ref[...] = anch_ref[...] * gate
        rowk = lax.broadcasted_iota(jnp.int32, (KP, 1), 0)
        g = jnp.clip(gate, EPS, 1.0 - EPS)
        t = g * jnp.log(2.0 * g) + (1.0 - g) * jnp.log(2.0 * (1.0 - g))
        akl_ref[...] = jnp.reshape(jnp.sum(jnp.where(rowk < K, t, 0.0)) / K, (1, 1))


def _p3_body(nx_ref, ax_ref, asg_ref):
    x = nx_ref[...]
    a = ax_ref[...]
    xy = lax.dot_general(x, a, (((1,), (0,)), ((), ())),
                         preferred_element_type=jnp.float32)
    sx = jnp.sum(x * x, axis=1, keepdims=True)
    sa = jnp.sum(a * a, axis=0, keepdims=True)
    d2 = (sx + sa) - 2.0 * xy
    lane = lax.broadcasted_iota(jnp.int32, d2.shape, 1)
    d2 = d2 + jnp.where(lane >= K, jnp.float32(3.0e38), jnp.float32(0.0))
    mn = jnp.min(d2, axis=1, keepdims=True)
    idx = jnp.min(jnp.where(d2 == mn, lane, jnp.int32(2 ** 30)),
                  axis=1, keepdims=True)
    asg_ref[...] = idx


def _p4_body(asg_ref, nf_ref, nx_ref, ax4_ref, we_ref, af0_ref,
             agg_ref, cnt_ref, af1_ref):
    i = pl.program_id(0)

    @pl.when(i == 0)
    def _():
        agg_ref[...] = jnp.zeros_like(agg_ref)
        cnt_ref[...] = jnp.zeros_like(cnt_ref)

    asg = asg_ref[...]
    lane = lax.broadcasted_iota(jnp.int32, (1024, KP), 1)
    m = (asg == lane).astype(jnp.float32)
    ue = lax.dot_general(ax4_ref[...], we_ref[...], (((1,), (0,)), ((), ())),
                         preferred_element_type=jnp.float32)
    ua = _dot2(m, ue, (((1,), (0,)), ((), ())))
    ve = lax.dot_general(nx_ref[...], we_ref[...], (((1,), (0,)), ((), ())),
                         preferred_element_type=jnp.float32)
    msg = nf_ref[...] + jax.nn.relu(ua - ve)
    rid = lax.broadcasted_iota(jnp.int32, (1024, 1), 0)
    valid = (i * 1024 + rid) < N
    dims = (((0,), (0,)), ((), ()))
    agg_ref[...] += _dot2(m, jnp.where(valid, msg, 0.0), dims)
    lane8 = lax.broadcasted_iota(jnp.int32, (1024, 8), 1)
    vcol = jnp.where((lane8 == 0) & valid, 1.0, 0.0)
    cnt_ref[...] += lax.dot_general(m, vcol, dims,
                                    preferred_element_type=jnp.float32)

    @pl.when(i == pl.num_programs(0) - 1)
    def _():
        cnt = cnt_ref[:, 0:1]
        af1_ref[...] = af0_ref[...] + agg_ref[...] / jnp.maximum(cnt, 1.0)


def _p56_body(c0_ref, c1_ref, af1_ref, ax4_ref, we_ref, wa_ref, wo_ref,
              asg_ref, nf_ref, nx_ref, out_ref, t2_ref, ue_ref, af2_ref):
    i = pl.program_id(0)

    @pl.when(i == 0)
    def _():
        af1 = af1_ref[...]
        term1 = lax.dot_general(c0_ref[...] + c1_ref[...], af1,
                                (((1,), (0,)), ((), ())), precision=HI,
                                preferred_element_type=jnp.float32)
        ue_ref[...] = lax.dot_general(ax4_ref[...], we_ref[...],
                                      (((1,), (0,)), ((), ())),
                                      preferred_element_type=jnp.float32)

        def dblock(j, _):
            ued = ue_ref[pl.ds(j * 8, 8), :]

            def schunk(mm, acc):
                ue_sc = ue_ref[pl.ds(mm * 128, 128), :]
                w = (c0_ref[pl.ds(j * 8, 8), pl.ds(mm * 128, 128)]
                     + c1_ref[pl.ds(j * 8, 8), pl.ds(mm * 128, 128)])
                sidx = lax.broadcasted_iota(jnp.int32, (8, 128), 1) + mm * 128
                w = jnp.where(sidx < K, w, 0.0)
                t = jax.nn.relu(ue_sc[None, :, :] - ued[:, None, :])
                return acc + jnp.sum(t * w[:, :, None], axis=1)

            acc = lax.fori_loop(0, KP // 128, schunk,
                                jnp.zeros((8, D), jnp.float32))
            t2_ref[pl.ds(j * 8, 8), :] = acc
            return 0

        lax.fori_loop(0, KP // 8, dblock, 0)
        x = af1 + term1 + t2_ref[...]
        af2_ref[...] = jax.nn.relu(
            lax.dot_general(x, wa_ref[...], (((1,), (0,)), ((), ())),
                            preferred_element_type=jnp.float32))

    @pl.when(i > 0)
    def _():
        asg = asg_ref[...]
        lane = lax.broadcasted_iota(jnp.int32, (1024, KP), 1)
        m = (asg == lane).astype(jnp.float32)
        pack = jnp.concatenate([af2_ref[...], ue_ref[...]], axis=1)
        g = _dot2(m, pack, (((1,), (0,)), ((), ())))
        ve = lax.dot_general(nx_ref[...], we_ref[...], (((1,), (0,)), ((), ())),
                             preferred_element_type=jnp.float32)
        gathered = g[:, :D] + jax.nn.relu(ve - g[:, D:])
        upd = jax.nn.relu(
            lax.dot_general(gathered, wo_ref[...], (((1,), (0,)), ((), ())),
                            preferred_element_type=jnp.float32))
        out_ref[...] = nf_ref[...] + upd


def _sc_hist_body(es_hbm, ed_hbm, asg_hbm, out_hbm,
                  src_v, dst_v, asg_v, pair_v, ones_v, buf_v, c_sh):
    c = lax.axis_index("c")
    s = lax.axis_index("s")
    wid = c * 16 + s

    def zb(i, _):
        buf_v[pl.ds(i * 16, 16)] = jnp.zeros((16,), jnp.float32)
        return 0
    lax.fori_loop(0, CBUF // 16, zb, 0)

    def zcp(t, _):
        pltpu.sync_copy(buf_v, c_sh.at[pl.ds(s * CSTRIPE + t * CBUF, CBUF)])
        return 0
    lax.fori_loop(0, CSTRIPE // CBUF, zcp, 0)
    plsc.subcore_barrier()

    base = wid * ET
    pltpu.sync_copy(es_hbm.at[pl.ds(base, ET)], src_v)
    pltpu.sync_copy(ed_hbm.at[pl.ds(base, ET)], dst_v)
    pltpu.sync_copy(asg_hbm, asg_v)

    def step(i, _):
        valid = i < (ET // 16)
        off = jnp.where(valid, i * 16, 0)
        sv = src_v[pl.ds(off, 16)]
        dv = dst_v[pl.ds(off, 16)]
        a_s = plsc.load_gather(asg_v, [sv])
        a_d = plsc.load_gather(asg_v, [dv])
        pair = jnp.where(valid, a_d * KP + a_s, jnp.int32(CFLAT - 1))
        ones = jnp.where(valid, jnp.float32(1.0), jnp.float32(0.0))
        ones16 = jnp.full((16,), 0.0, jnp.float32) + ones
        pair_v[pl.ds(i * 16, 16)] = pair
        ones_v[pl.ds(i * 16, 16)] = ones16
        return 0

    lax.fori_loop(0, ETP // 16, step, 0)
    pltpu.sync_copy(ones_v, c_sh.at[pair_v], add=True)
    plsc.subcore_barrier()

    def xcp(t, _):
        pltpu.sync_copy(c_sh.at[pl.ds(s * CSTRIPE + t * CBUF, CBUF)], buf_v)
        pltpu.sync_copy(buf_v, out_hbm.at[c, pl.ds(s * CSTRIPE + t * CBUF, CBUF)])
        return 0
    lax.fori_loop(0, CSTRIPE // CBUF, xcp, 0)


def _sc_hist(edge_src, edge_dst, assign_flat):
    mesh = plsc.VectorSubcoreMesh(core_axis_name="c", subcore_axis_name="s")
    f = pl.kernel(
        _sc_hist_body,
        out_type=jax.ShapeDtypeStruct((2, CFLAT), jnp.float32),
        mesh=mesh,
        compiler_params=pltpu.CompilerParams(needs_layout_passes=False),
        scratch_types=[
            pltpu.VMEM((ET,), jnp.int32),
            pltpu.VMEM((ET,), jnp.int32),
            pltpu.VMEM((NP,), jnp.int32),
            pltpu.VMEM((ETP,), jnp.int32),
            pltpu.VMEM((ETP,), jnp.float32),
            pltpu.VMEM((CBUF,), jnp.float32),
            pltpu.VMEM_SHARED((CFLAT,), jnp.float32),
        ],
    )
    return f(edge_src, edge_dst, assign_flat)


def kernel(node_x, node_features, edge_index, batch, W_score, W_edge, W_anchor, W_out):
    f32 = jnp.float32
    nf_pad = jnp.pad(node_features, ((0, NP - N), (0, 0)))
    nx4 = jnp.pad(node_x, ((0, NP - N), (0, 1)))
    we4 = jnp.pad(W_edge, ((0, 1), (0, 0)))
    es = edge_index[0].astype(jnp.int32)
    ed = edge_index[1].astype(jnp.int32)

    score, rank80, nkl = pl.pallas_call(
        _p1_body,
        out_shape=[jax.ShapeDtypeStruct((NP, 1), f32),
                   jax.ShapeDtypeStruct((80, 128), jnp.int32),
                   jax.ShapeDtypeStruct((1, 1), f32)],
    )(nf_pad, W_score)
    rank2 = rank80.reshape(NP, 1)

    payx = jnp.concatenate([nx4[:, :3], score, jnp.zeros((NP, 4), f32)], axis=1)
    af0, anchx, akl = pl.pallas_call(
        _p2b_body,
        grid=(NP // 1024,),
        in_specs=[pl.BlockSpec((1024, 1), lambda i: (i, 0)),
                  pl.BlockSpec((1024, D), lambda i: (i, 0)),
                  pl.BlockSpec((1024, 8), lambda i: (i, 0))],
        out_specs=[pl.BlockSpec((KP, D), lambda i: (0, 0)),
                   pl.BlockSpec((KP, 8), lambda i: (0, 0)),
                   pl.BlockSpec((1, 1), lambda i: (0, 0))],
        out_shape=[jax.ShapeDtypeStruct((KP, D), f32),
                   jax.ShapeDtypeStruct((KP, 8), f32),
                   jax.ShapeDtypeStruct((1, 1), f32)],
    )(rank2, nf_pad, payx)

    ax4 = jnp.concatenate([anchx[:, :3], jnp.zeros((KP, 1), f32)], axis=1)
    axT4 = ax4.T

    assign = pl.pallas_call(
        _p3_body,
        grid=(NP // 2048,),
        in_specs=[pl.BlockSpec((2048, 4), lambda i: (i, 0)),
                  pl.BlockSpec((4, KP), lambda i: (0, 0))],
        out_specs=pl.BlockSpec((2048, 1), lambda i: (i, 0)),
        out_shape=jax.ShapeDtypeStruct((NP, 1), jnp.int32),
    )(nx4, axT4)

    c2 = _sc_hist(es, ed, assign.reshape(NP))
    c0 = c2[0].reshape(KP, KP)
    c1 = c2[1].reshape(KP, KP)

    _, _, af1 = pl.pallas_call(
        _p4_body,
        grid=(NP // 1024,),
        in_specs=[pl.BlockSpec((1024, 1), lambda i: (i, 0)),
                  pl.BlockSpec((1024, D), lambda i: (i, 0)),
                  pl.BlockSpec((1024, 4), lambda i: (i, 0)),
                  pl.BlockSpec((KP, 4), lambda i: (0, 0)),
                  pl.BlockSpec((4, D), lambda i: (0, 0)),
                  pl.BlockSpec((KP, D), lambda i: (0, 0))],
        out_specs=[pl.BlockSpec((KP, D), lambda i: (0, 0)),
                   pl.BlockSpec((KP, 8), lambda i: (0, 0)),
                   pl.BlockSpec((KP, D), lambda i: (0, 0))],
        out_shape=[jax.ShapeDtypeStruct((KP, D), f32),
                   jax.ShapeDtypeStruct((KP, 8), f32),
                   jax.ShapeDtypeStruct((KP, D), f32)],
    )(assign, nf_pad, nx4, ax4, we4, af0)

    blk = lambda i: (jnp.maximum(i - 1, 0), 0)
    cst = lambda i: (0, 0)
    nf_out = pl.pallas_call(
        _p56_body,
        grid=(NP // 1024 + 1,),
        in_specs=[pl.BlockSpec((KP, KP), cst),
                  pl.BlockSpec((KP, KP), cst),
                  pl.BlockSpec((KP, D), cst),
                  pl.BlockSpec((KP, 4), cst),
                  pl.BlockSpec((4, D), cst),
                  pl.BlockSpec((D, D), cst),
                  pl.BlockSpec((D, D), cst),
                  pl.BlockSpec((1024, 1), blk),
                  pl.BlockSpec((1024, D), blk),
                  pl.BlockSpec((1024, 4), blk)],
        out_specs=pl.BlockSpec((1024, D), blk),
        out_shape=jax.ShapeDtypeStruct((NP, D), f32),
        scratch_shapes=[pltpu.VMEM((KP, D), f32), pltpu.VMEM((KP, D), f32),
                        pltpu.VMEM((KP, D), f32)],
    )(c0, c1, af1, ax4, we4, W_anchor, W_out, assign, nf_pad, nx4)

    return (nf_out[:N], akl[0, 0], nkl[0, 0])
